# baseline (speedup 1.0000x reference)
import functools

import jax
import jax.numpy as jnp
from jax import lax
from jax.experimental import pallas as pl
from jax.experimental.pallas import tpu as pltpu
from jax.experimental.pallas import tpu_sc as plsc

VOCAB = 1000000
DIM = 64
SEQ = 200
BATCH = 4096

NC = 2
NS = 16
NW = NC * NS

B = SEQ * BATCH
PER_W = B // NW
CH = 128
NCH = PER_W // CH
NBUF = 8


def _emb_kernel(table_hbm, idx_hbm, out_hbm, idx_v, rows_v, gsem, osem):
    wid = lax.axis_index("s") * NC + lax.axis_index("c")
    base = wid * PER_W

    pltpu.sync_copy(idx_hbm.at[wid], idx_v)

    for b in range(NBUF):
        pltpu.async_copy(table_hbm.at[idx_v.at[b]], rows_v.at[b], gsem)

    def body(j, _):
        b = lax.rem(j, NBUF)
        pltpu.make_async_copy(
            table_hbm.at[idx_v.at[b]], rows_v.at[b], gsem
        ).wait()
        pltpu.async_copy(
            rows_v.at[b], out_hbm.at[pl.ds(base + j * CH, CH)], osem
        ).wait()

        @pl.when(j + NBUF < NCH)
        def _():
            pltpu.async_copy(
                table_hbm.at[idx_v.at[j + NBUF]], rows_v.at[b], gsem
            )

        return 0

    lax.fori_loop(0, NCH, body, 0)


@jax.jit
def _emb(table, idx3):
    mesh = plsc.VectorSubcoreMesh(
        core_axis_name="c", subcore_axis_name="s",
        num_cores=NC, num_subcores=NS,
    )
    run = pl.kernel(
        _emb_kernel,
        out_type=jax.ShapeDtypeStruct((B, DIM), jnp.float32),
        mesh=mesh,
        scratch_types=[
            pltpu.VMEM((NCH, CH), jnp.int32),
            pltpu.VMEM((NBUF, CH, DIM), jnp.float32),
            pltpu.SemaphoreType.DMA,
            pltpu.SemaphoreType.DMA,
        ],
        compiler_params=pltpu.CompilerParams(use_tc_tiling_on_sc=False),
    )
    return run(table, idx3)


def kernel(src_input, table):
    idx = src_input.reshape(B).astype(jnp.int32)
    idx3 = idx.reshape(NW, NCH, CH)
    out = _emb(table, idx3)
    return out.reshape(SEQ, BATCH, DIM)


# speedup vs baseline: 1.0010x; 1.0010x over previous
import functools

import jax
import jax.numpy as jnp
from jax import lax
from jax.experimental import pallas as pl
from jax.experimental.pallas import tpu as pltpu
from jax.experimental.pallas import tpu_sc as plsc

VOCAB = 1000000
DIM = 64
SEQ = 200
BATCH = 4096

NC = 2
NS = 16
NW = NC * NS

B = SEQ * BATCH
PER_W = B // NW
CH = 128
NCH = PER_W // CH
NBUF = 8
KAHEAD = 4


def _emb_kernel(table_hbm, idx_hbm, out_hbm, idx_v, rows_v, gsem, osem):
    wid = lax.axis_index("s") * NC + lax.axis_index("c")
    base = wid * PER_W

    pltpu.sync_copy(idx_hbm.at[wid], idx_v)

    for b in range(KAHEAD):
        pltpu.async_copy(table_hbm.at[idx_v.at[b]], rows_v.at[b], gsem)

    def body(j, _):
        b = lax.rem(j, NBUF)
        pltpu.make_async_copy(
            table_hbm.at[idx_v.at[b]], rows_v.at[b], gsem
        ).wait()
        pltpu.async_copy(
            rows_v.at[b], out_hbm.at[pl.ds(base + j * CH, CH)], osem
        )

        @pl.when(j >= NBUF - KAHEAD)
        def _():
            pltpu.make_async_copy(
                rows_v.at[b], out_hbm.at[pl.ds(base, CH)], osem
            ).wait()

        @pl.when(j + KAHEAD < NCH)
        def _():
            bn = lax.rem(j + KAHEAD, NBUF)
            pltpu.async_copy(
                table_hbm.at[idx_v.at[j + KAHEAD]], rows_v.at[bn], gsem
            )

        return 0

    lax.fori_loop(0, NCH, body, 0)

    for _ in range(NBUF - KAHEAD):
        pltpu.make_async_copy(
            rows_v.at[0], out_hbm.at[pl.ds(base, CH)], osem
        ).wait()


@jax.jit
def _emb(table, idx3):
    mesh = plsc.VectorSubcoreMesh(
        core_axis_name="c", subcore_axis_name="s",
        num_cores=NC, num_subcores=NS,
    )
    run = pl.kernel(
        _emb_kernel,
        out_type=jax.ShapeDtypeStruct((B, DIM), jnp.float32),
        mesh=mesh,
        scratch_types=[
            pltpu.VMEM((NCH, CH), jnp.int32),
            pltpu.VMEM((NBUF, CH, DIM), jnp.float32),
            pltpu.SemaphoreType.DMA,
            pltpu.SemaphoreType.DMA,
        ],
        compiler_params=pltpu.CompilerParams(use_tc_tiling_on_sc=False),
    )
    return run(table, idx3)


def kernel(src_input, table):
    idx = src_input.reshape(B).astype(jnp.int32)
    idx3 = idx.reshape(NW, NCH, CH)
    out = _emb(table, idx3)
    return out.reshape(SEQ, BATCH, DIM)
